# initial kernel scaffold (unmeasured)
import jax
import jax.numpy as jnp
from jax import lax
from jax.experimental import pallas as pl
from jax.experimental.pallas import tpu as pltpu


def kernel(
    x,
):
    def body(*refs):
        pass

    out_shape = jax.ShapeDtypeStruct(..., jnp.float32)
    return pl.pallas_call(body, out_shape=out_shape)(...)


# baseline (device time: 70506 ns/iter reference)
import numpy as np
import jax
import jax.numpy as jnp
from jax import lax
from jax.experimental import pallas as pl
from jax.experimental.pallas import tpu as pltpu

N_DEV = 16


def _bitonic_sort(v, k_lo=1, k_hi=None, flip=None):
    m, n = v.shape
    log_m = int(m).bit_length() - 1
    if k_hi is None:
        k_hi = log_m
    for k in range(k_lo, k_hi + 1):
        blk = 1 << k
        for j in range(k - 1, -1, -1):
            s = 1 << j
            g = m // (2 * s)
            v4 = v.reshape(g, 2, s, n)
            a = v4[:, 0, :, :]
            b = v4[:, 1, :, :]
            lo = jnp.minimum(a, b)
            hi = jnp.maximum(a, b)
            gidx = lax.broadcasted_iota(jnp.int32, (g, 1, 1), 0)
            asc = (gidx * (2 * s) & blk) == 0
            if flip is not None:
                asc = jnp.logical_xor(asc, flip)
            na = jnp.where(asc, lo, hi)
            nb = jnp.where(asc, hi, lo)
            v = jnp.concatenate([na[:, None], nb[:, None]], axis=1).reshape(m, n)
    return v


def kernel(x):
    m_per, n = x.shape

    def body(x_ref, out_ref, gather_ref, sort_ref, send_sems, recv_sems, copy_sem):
        my = lax.axis_index("i")

        barrier = pltpu.get_barrier_semaphore()
        for k in range(1, N_DEV):
            tgt = lax.rem(my + k, N_DEV)
            pl.semaphore_signal(
                barrier, inc=1, device_id=(tgt,),
                device_id_type=pl.DeviceIdType.MESH,
            )
        pl.semaphore_wait(barrier, N_DEV - 1)

        local = pltpu.make_async_copy(x_ref, gather_ref.at[my], copy_sem)
        local.start()

        sends = []
        for k in range(1, N_DEV):
            tgt = lax.rem(my + k, N_DEV)
            rdma = pltpu.make_async_remote_copy(
                src_ref=x_ref,
                dst_ref=gather_ref.at[my],
                send_sem=send_sems.at[k - 1],
                recv_sem=recv_sems.at[my],
                device_id=(tgt,),
                device_id_type=pl.DeviceIdType.MESH,
            )
            rdma.start()
            sends.append(rdma)

        local.wait()
        for k in range(1, N_DEV):
            src = lax.rem(my + k, N_DEV)
            recv = pltpu.make_async_remote_copy(
                src_ref=x_ref,
                dst_ref=gather_ref.at[src],
                send_sem=send_sems.at[k - 1],
                recv_sem=recv_sems.at[src],
                device_id=(src,),
                device_id_type=pl.DeviceIdType.MESH,
            )
            recv.wait_recv()
        for rdma in sends:
            rdma.wait_send()

        v = gather_ref[...].reshape(N_DEV * m_per, n)
        sort_ref[...] = _bitonic_sort(v)
        out_ref[...] = sort_ref[pl.ds(my * m_per, m_per), :]

    return pl.pallas_call(
        body,
        out_shape=jax.ShapeDtypeStruct((m_per, n), x.dtype),
        in_specs=[pl.BlockSpec(memory_space=pltpu.VMEM)],
        out_specs=pl.BlockSpec(memory_space=pltpu.VMEM),
        scratch_shapes=[
            pltpu.VMEM((N_DEV, m_per, n), x.dtype),
            pltpu.VMEM((N_DEV * m_per, n), x.dtype),
            pltpu.SemaphoreType.DMA((N_DEV - 1,)),
            pltpu.SemaphoreType.DMA((N_DEV,)),
            pltpu.SemaphoreType.DMA,
        ],
        compiler_params=pltpu.CompilerParams(collective_id=0),
    )(x)


# device time: 44052 ns/iter; 1.6005x vs baseline; 1.6005x over previous
import numpy as np
import jax
import jax.numpy as jnp
from jax import lax
from jax.experimental import pallas as pl
from jax.experimental.pallas import tpu as pltpu

N_DEV = 16


def _bitonic_sort(v, k_lo=1, k_hi=None, flip=None):
    m, n = v.shape
    log_m = int(m).bit_length() - 1
    if k_hi is None:
        k_hi = log_m
    for k in range(k_lo, k_hi + 1):
        blk = 1 << k
        for j in range(k - 1, -1, -1):
            s = 1 << j
            g = m // (2 * s)
            v4 = v.reshape(g, 2, s, n)
            a = v4[:, 0, :, :]
            b = v4[:, 1, :, :]
            lo = jnp.minimum(a, b)
            hi = jnp.maximum(a, b)
            gidx = lax.broadcasted_iota(jnp.int32, (g, 1, 1), 0)
            asc = (gidx * (2 * s) & blk) == 0
            if flip is not None:
                asc = jnp.logical_xor(asc, flip)
            na = jnp.where(asc, lo, hi)
            nb = jnp.where(asc, hi, lo)
            v = jnp.concatenate([na[:, None], nb[:, None]], axis=1).reshape(m, n)
    return v


def kernel(x):
    m_per, n = x.shape

    def body(x_ref, out_ref, send_ref, gather_ref, sort_ref, send_sems,
             recv_sems, copy_sem):
        my = lax.axis_index("i")

        flip = lax.rem(my, 2) == 1
        send_ref[...] = _bitonic_sort(x_ref[...], k_hi=7, flip=flip)

        barrier = pltpu.get_barrier_semaphore()
        for k in range(1, N_DEV):
            tgt = lax.rem(my + k, N_DEV)
            pl.semaphore_signal(
                barrier, inc=1, device_id=(tgt,),
                device_id_type=pl.DeviceIdType.MESH,
            )
        pl.semaphore_wait(barrier, N_DEV - 1)

        local = pltpu.make_async_copy(send_ref, gather_ref.at[my], copy_sem)
        local.start()

        sends = []
        for k in range(1, N_DEV):
            tgt = lax.rem(my + k, N_DEV)
            rdma = pltpu.make_async_remote_copy(
                src_ref=send_ref,
                dst_ref=gather_ref.at[my],
                send_sem=send_sems.at[k - 1],
                recv_sem=recv_sems.at[my],
                device_id=(tgt,),
                device_id_type=pl.DeviceIdType.MESH,
            )
            rdma.start()
            sends.append(rdma)

        local.wait()
        for k in range(1, N_DEV):
            src = lax.rem(my + k, N_DEV)
            recv = pltpu.make_async_remote_copy(
                src_ref=send_ref,
                dst_ref=gather_ref.at[src],
                send_sem=send_sems.at[k - 1],
                recv_sem=recv_sems.at[src],
                device_id=(src,),
                device_id_type=pl.DeviceIdType.MESH,
            )
            recv.wait_recv()
        for rdma in sends:
            rdma.wait_send()

        v = gather_ref[...].reshape(N_DEV * m_per, n)
        sort_ref[...] = _bitonic_sort(v, k_lo=8)
        out_ref[...] = sort_ref[pl.ds(my * m_per, m_per), :]

    return pl.pallas_call(
        body,
        out_shape=jax.ShapeDtypeStruct((m_per, n), x.dtype),
        in_specs=[pl.BlockSpec(memory_space=pltpu.VMEM)],
        out_specs=pl.BlockSpec(memory_space=pltpu.VMEM),
        scratch_shapes=[
            pltpu.VMEM((m_per, n), x.dtype),
            pltpu.VMEM((N_DEV, m_per, n), x.dtype),
            pltpu.VMEM((N_DEV * m_per, n), x.dtype),
            pltpu.SemaphoreType.DMA((N_DEV - 1,)),
            pltpu.SemaphoreType.DMA((N_DEV,)),
            pltpu.SemaphoreType.DMA,
        ],
        compiler_params=pltpu.CompilerParams(collective_id=0),
    )(x)


# device time: 25173 ns/iter; 2.8009x vs baseline; 1.7500x over previous
import numpy as np
import jax
import jax.numpy as jnp
from jax import lax
from jax.experimental import pallas as pl
from jax.experimental.pallas import tpu as pltpu

N_DEV = 16


def _bitonic_sort(v, k_lo=1, k_hi=None, flip=None):
    m, n = v.shape
    log_m = int(m).bit_length() - 1
    if k_hi is None:
        k_hi = log_m
    for k in range(k_lo, k_hi + 1):
        blk = 1 << k
        for j in range(k - 1, -1, -1):
            s = 1 << j
            g = m // (2 * s)
            v4 = v.reshape(g, 2, s, n)
            a = v4[:, 0, :, :]
            b = v4[:, 1, :, :]
            lo = jnp.minimum(a, b)
            hi = jnp.maximum(a, b)
            gidx = lax.broadcasted_iota(jnp.int32, (g, 1, 1), 0)
            asc = (gidx * (2 * s) & blk) == 0
            if flip is not None:
                asc = jnp.logical_xor(asc, flip)
            na = jnp.where(asc, lo, hi)
            nb = jnp.where(asc, hi, lo)
            v = jnp.concatenate([na[:, None], nb[:, None]], axis=1).reshape(m, n)
    return v


def _merge_packed(p):
    m, n2 = p.shape
    for k in (8, 9):
        blk = 1 << k
        for j in range(k - 1, -1, -1):
            s = 1 << j
            g = m // (2 * s)
            v4 = p.reshape(g, 2, s, n2)
            a, b = v4[:, 0], v4[:, 1]
            lo, hi = jnp.minimum(a, b), jnp.maximum(a, b)
            gidx = lax.broadcasted_iota(jnp.int32, (g, 1, 1), 0)
            asc = (gidx * (2 * s) & blk) == 0
            na, nb = jnp.where(asc, lo, hi), jnp.where(asc, hi, lo)
            p = jnp.concatenate([na[:, None], nb[:, None]], axis=1).reshape(m, n2)
    lane = lax.broadcasted_iota(jnp.int32, (1, 1, n2), 2)
    asc_l = lane < (n2 // 2)
    for j in range(9, -1, -1):
        s = 1 << j
        g = m // (2 * s)
        v4 = p.reshape(g, 2, s, n2)
        a, b = v4[:, 0], v4[:, 1]
        lo, hi = jnp.minimum(a, b), jnp.maximum(a, b)
        na, nb = jnp.where(asc_l, lo, hi), jnp.where(asc_l, hi, lo)
        p = jnp.concatenate([na[:, None], nb[:, None]], axis=1).reshape(m, n2)
    a, b = p[:, : n2 // 2], p[:, n2 // 2:]
    p = jnp.concatenate([jnp.minimum(a, b), jnp.maximum(a, b)], axis=1)
    for j in range(9, -1, -1):
        s = 1 << j
        g = m // (2 * s)
        v4 = p.reshape(g, 2, s, n2)
        a, b = v4[:, 0], v4[:, 1]
        p = jnp.concatenate(
            [jnp.minimum(a, b)[:, None], jnp.maximum(a, b)[:, None]], axis=1
        ).reshape(m, n2)
    return p


def kernel(x):
    m_per, n = x.shape

    def body(x_ref, out_ref, send_ref, gather_ref, sort_ref, send_sems,
             recv_sems, copy_sem):
        my = lax.axis_index("i")

        flip = lax.rem(my, 2) == 1
        send_ref[...] = _bitonic_sort(x_ref[...], k_hi=7, flip=flip)

        barrier = pltpu.get_barrier_semaphore()
        for k in range(1, N_DEV):
            tgt = lax.rem(my + k, N_DEV)
            pl.semaphore_signal(
                barrier, inc=1, device_id=(tgt,),
                device_id_type=pl.DeviceIdType.MESH,
            )
        pl.semaphore_wait(barrier, N_DEV - 1)

        local = pltpu.make_async_copy(send_ref, gather_ref.at[my], copy_sem)
        local.start()

        sends = []
        for k in range(1, N_DEV):
            tgt = lax.rem(my + k, N_DEV)
            rdma = pltpu.make_async_remote_copy(
                src_ref=send_ref,
                dst_ref=gather_ref.at[my],
                send_sem=send_sems.at[k - 1],
                recv_sem=recv_sems.at[my],
                device_id=(tgt,),
                device_id_type=pl.DeviceIdType.MESH,
            )
            rdma.start()
            sends.append(rdma)

        local.wait()
        for k in range(1, N_DEV):
            src = lax.rem(my + k, N_DEV)
            recv = pltpu.make_async_remote_copy(
                src_ref=send_ref,
                dst_ref=gather_ref.at[src],
                send_sem=send_sems.at[k - 1],
                recv_sem=recv_sems.at[src],
                device_id=(src,),
                device_id_type=pl.DeviceIdType.MESH,
            )
            recv.wait_recv()
        for rdma in sends:
            rdma.wait_send()

        g2 = gather_ref[...].reshape(N_DEV * m_per, n)
        half = N_DEV * m_per // 2
        p = jnp.concatenate([g2[:half], g2[half:]], axis=1)
        sort_ref[...] = _merge_packed(p)
        r0 = lax.rem(my, N_DEV // 2) * m_per

        @pl.when(my < N_DEV // 2)
        def _():
            out_ref[...] = sort_ref[pl.ds(r0, m_per), 0:n]

        @pl.when(my >= N_DEV // 2)
        def _():
            out_ref[...] = sort_ref[pl.ds(r0, m_per), n:2 * n]

    return pl.pallas_call(
        body,
        out_shape=jax.ShapeDtypeStruct((m_per, n), x.dtype),
        in_specs=[pl.BlockSpec(memory_space=pltpu.VMEM)],
        out_specs=pl.BlockSpec(memory_space=pltpu.VMEM),
        scratch_shapes=[
            pltpu.VMEM((m_per, n), x.dtype),
            pltpu.VMEM((N_DEV, m_per, n), x.dtype),
            pltpu.VMEM((N_DEV * m_per // 2, 2 * n), x.dtype),
            pltpu.SemaphoreType.DMA((N_DEV - 1,)),
            pltpu.SemaphoreType.DMA((N_DEV,)),
            pltpu.SemaphoreType.DMA,
        ],
        compiler_params=pltpu.CompilerParams(collective_id=0),
    )(x)
